# baseline (speedup 1.0000x reference)
import functools

import jax
import jax.numpy as jnp
import numpy as np
from jax import lax
from jax.experimental import pallas as pl
from jax.experimental.pallas import tpu as pltpu
from jax.experimental.pallas import tpu_sc as plsc

_ANCHORS = np.array(
    [[1.3221 / 13.0, 1.73145 / 13.0],
     [3.19275 / 13.0, 4.00944 / 13.0],
     [5.05587 / 13.0, 8.09892 / 13.0],
     [9.47112 / 13.0, 4.84053 / 13.0],
     [11.2364 / 13.0, 10.0071 / 13.0]], dtype=np.float32)
_THRESH = 0.6
_PRIOR_ITER = 12800

_A = 5
_K = 50
_KP = 64


def _per_truth(x1, y1, x2, y2, w_grid, h_grid):
    cw = x2 - x1
    ch = y2 - y1
    a1 = cw * ch
    best_iou = jnp.zeros_like(cw)
    best_ind = jnp.zeros_like(cw)
    for a in range(_A):
        aw = float(_ANCHORS[a, 0])
        ah = float(_ANCHORS[a, 1])
        a2 = float(np.float32(_ANCHORS[a, 0]) * np.float32(_ANCHORS[a, 1]))
        inter = jnp.minimum(cw, aw) * jnp.minimum(ch, ah)
        union = jnp.clip(a1 + a2 - inter, 1e-12, None)
        iou = inter / union
        upd = iou > best_iou
        best_ind = jnp.where(upd, float(a), best_ind)
        best_iou = jnp.where(upd, iou, best_iou)
    dx = (x1 + x2) / 2.0 * w_grid
    dy = (y1 + y2) / 2.0 * h_grid
    gxk = jnp.ceil(dx) - 1.0
    gyk = jnp.ceil(dy) - 1.0
    pos = gyk * w_grid + gxk
    return pos, best_ind, best_iou, cw, ch, dx - gxk, dy - gyk


def _meta_body(tp_ref, out_ref, *, grid_h, grid_w, t_ch, c_ch):
    tp = tp_ref[...]
    x1 = tp[:, 0, :]
    y1 = tp[:, 1, :]
    x2 = tp[:, 2, :]
    y2 = tp[:, 3, :]
    pos, ind, miou, cw, ch, fx, fy = _per_truth(
        x1, y1, x2, y2, float(grid_w), float(grid_h))
    valid = miou != 0.0
    iota_j = lax.broadcasted_iota(jnp.int32, (1, _KP, _KP), 1)
    iota_m = lax.broadcasted_iota(jnp.int32, (1, _KP, _KP), 2)
    pj = pos[:, :, None]
    pm = pos[:, None, :]
    ij = ind[:, :, None]
    im = ind[:, None, :]
    mj = miou[:, :, None]
    mm = miou[:, None, :]
    beats = (pj == pm) & (ij == im) & (
        (mm > mj) | ((mm == mj) & (iota_m < iota_j)))
    winner = valid & jnp.logical_not(jnp.any(beats, axis=2))

    rdx = -jnp.log(1.0 / fx - 1.0)
    rdy = -jnp.log(1.0 / fy - 1.0)
    aw_sel = jnp.zeros_like(ind)
    ah_sel = jnp.zeros_like(ind)
    for a in range(_A):
        hit = ind == float(a)
        aw_sel = jnp.where(hit, float(_ANCHORS[a, 0]), aw_sel)
        ah_sel = jnp.where(hit, float(_ANCHORS[a, 1]), ah_sel)
    t2t = jnp.log(cw) / aw_sel
    t3t = jnp.log(ch) / ah_sel
    cf2 = 2.0 - cw * ch

    fb = pos * float(t_ch) + ind * float(c_ch)
    zero = jnp.zeros_like(fb)
    rows = [jnp.where(winner, fb, zero), jnp.where(winner, cf2, zero),
            jnp.where(winner, rdx, zero), jnp.where(winner, rdy, zero),
            jnp.where(winner, t2t, zero), jnp.where(winner, t3t, zero),
            zero, zero]
    out_ref[...] = jnp.concatenate([r[:, None, :] for r in rows], axis=1)


def _make_sc_coord(b_total, elems_per_batch):
    info = plsc.get_sparse_core_info()
    nw = info.num_cores * info.num_subcores
    b_per_w = b_total // nw
    mesh = plsc.VectorSubcoreMesh(core_axis_name="c", subcore_axis_name="s")

    @functools.partial(
        pl.kernel, mesh=mesh,
        out_type=jax.ShapeDtypeStruct((b_total, 16), jnp.float32),
        scratch_types=[
            pltpu.VMEM((8, _KP), jnp.float32),
            pltpu.VMEM((4, _KP), jnp.int32),
            pltpu.VMEM((4, _KP), jnp.float32),
            pltpu.VMEM((16,), jnp.float32),
            pltpu.SemaphoreType.DMA,
        ],
    )
    def sc_coord(xflat_hbm, meta_hbm, out_hbm, meta_v, idx_v, vals_v,
                 acc_v, sem):
        wid = lax.axis_index("s") * info.num_cores + lax.axis_index("c")
        for bi in range(b_per_w):
            b = wid * b_per_w + bi
            pltpu.sync_copy(meta_hbm.at[b], meta_v)
            base_b = b * elems_per_batch
            for c in range(4):
                for chk in range(_KP // 16):
                    fb = meta_v[0, pl.ds(chk * 16, 16)]
                    idx_v[c, pl.ds(chk * 16, 16)] = (
                        fb.astype(jnp.int32) + (base_b + c))
            for c in range(4):
                pltpu.async_copy(
                    xflat_hbm.at[idx_v.at[c]], vals_v.at[c], sem).wait()
            acc = jnp.zeros((16,), jnp.float32)
            for chk in range(_KP // 16):
                sl = pl.ds(chk * 16, 16)
                wcf2 = meta_v[1, sl]
                d0 = vals_v[0, sl] - meta_v[2, sl]
                d1 = vals_v[1, sl] - meta_v[3, sl]
                d2 = vals_v[2, sl] - meta_v[4, sl]
                d3 = vals_v[3, sl] - meta_v[5, sl]
                acc = acc + wcf2 * (d0 * d0 + d1 * d1 + d2 * d2 + d3 * d3)
            acc_v[...] = acc
            pltpu.sync_copy(acc_v, out_hbm.at[b])

    return sc_coord


def _dense_body(x_ref, tj_ref, out_ref, *, grid_h, grid_w):
    hw = grid_h * grid_w
    x = x_ref[0]
    tj = tj_ref[0]

    posj, indj, miouj = _per_truth(
        tj[:, 0:1], tj[:, 1:2], tj[:, 2:3], tj[:, 3:4],
        float(grid_w), float(grid_h))[:3]
    validj = miouj != 0.0

    cell = lax.broadcasted_iota(jnp.int32, (1, hw), 1)
    gx = (cell % grid_w).astype(jnp.float32)
    gy = (cell // grid_w).astype(jnp.float32)
    cellf = cell.astype(jnp.float32)

    tx1 = tj[:, 0:1]
    ty1 = tj[:, 1:2]
    tx2 = tj[:, 2:3]
    ty2 = tj[:, 3:4]
    a2t = 0.375 * ((tx2 - tx1) * (ty2 - ty1))

    acc_prior = jnp.float32(0.0)
    acc_noobj = jnp.float32(0.0)
    for a in range(_A):
        base = a * 5
        t0 = x[base + 0:base + 1, :]
        t1 = x[base + 1:base + 2, :]
        t2 = x[base + 2:base + 3, :]
        t3 = x[base + 3:base + 4, :]
        t4 = x[base + 4:base + 5, :]
        aw = float(_ANCHORS[a, 0])
        ah = float(_ANCHORS[a, 1])
        c0 = (1.0 / (1.0 + jnp.exp(-t0)) + gx) / float(grid_w)
        c1 = (1.0 / (1.0 + jnp.exp(-t1)) + gy) / float(grid_h)
        wa = jnp.exp(t2) * aw
        ha = jnp.exp(t3) * ah
        bx1 = c0 - wa / 2.0
        bx2 = c0 + wa / 2.0
        by1 = c1 - ha / 2.0
        by2 = c1 + ha / 2.0
        a1 = (bx2 - bx1) * (by2 - by1)
        acc_prior += jnp.sum((wa - aw) ** 2) + jnp.sum((ha - ah) ** 2)
        ix = jnp.clip(jnp.minimum(bx2, tx2) - jnp.maximum(bx1, tx1), 0.0, None)
        iy = jnp.clip(jnp.minimum(by2, ty2) - jnp.maximum(by1, ty1), 0.0, None)
        inter = ix * iy
        noobj = jnp.all(inter < (0.375 * a1 + a2t), axis=0, keepdims=True)
        match = (posj == cellf) & (indj == float(a)) & validj
        anymatch = jnp.any(match, axis=0, keepdims=True)
        acc_noobj += jnp.sum(
            jnp.where(noobj & jnp.logical_not(anymatch), t4, 0.0) ** 2)

    out_ref[0] = jnp.concatenate(
        [acc_prior.reshape(1, 1), acc_noobj.reshape(1, 1)], axis=1)


def kernel(output, truths, iteration):
    b, grid_h, grid_w, t = output.shape
    hw = grid_h * grid_w
    c_ch = t // _A
    k = truths.shape[1]

    tp = jnp.pad(truths, ((0, 0), (0, _KP - k), (0, 0)))
    tpt = jnp.transpose(tp, (0, 2, 1))
    meta = pl.pallas_call(
        functools.partial(_meta_body, grid_h=grid_h, grid_w=grid_w,
                          t_ch=t, c_ch=c_ch),
        out_shape=jax.ShapeDtypeStruct((b, 8, _KP), jnp.float32),
    )(tpt)

    xflat = output.reshape(-1)
    coord_parts = _make_sc_coord(b, hw * t)(xflat, meta)

    xt = output.reshape(b, hw, _A, c_ch)[:, :, :, 0:5]
    xt = jnp.transpose(xt, (0, 2, 3, 1)).reshape(b, _A * 5, hw)
    parts = pl.pallas_call(
        functools.partial(_dense_body, grid_h=grid_h, grid_w=grid_w),
        grid=(b,),
        in_specs=[
            pl.BlockSpec((1, _A * 5, hw), lambda i: (i, 0, 0)),
            pl.BlockSpec((1, k, 5), lambda i: (i, 0, 0)),
        ],
        out_specs=pl.BlockSpec((1, 1, 2), lambda i: (i, 0, 0)),
        out_shape=jax.ShapeDtypeStruct((b, 1, 2), jnp.float32),
    )(xt, truths)

    sums = jnp.sum(parts, axis=(0, 1))
    prior = jnp.where(iteration < _PRIOR_ITER, sums[0], jnp.float32(0.0))
    return prior + sums[1] + jnp.sum(coord_parts)


# speedup vs baseline: 1.1783x; 1.1783x over previous
import functools

import jax
import jax.numpy as jnp
import numpy as np
from jax import lax
from jax.experimental import pallas as pl
from jax.experimental.pallas import tpu as pltpu
from jax.experimental.pallas import tpu_sc as plsc

_ANCHORS = np.array(
    [[1.3221 / 13.0, 1.73145 / 13.0],
     [3.19275 / 13.0, 4.00944 / 13.0],
     [5.05587 / 13.0, 8.09892 / 13.0],
     [9.47112 / 13.0, 4.84053 / 13.0],
     [11.2364 / 13.0, 10.0071 / 13.0]], dtype=np.float32)
_THRESH = 0.6
_PRIOR_ITER = 12800

_A = 5
_K = 50
_KP = 64
_HP = jax.lax.Precision.HIGHEST


def _per_truth(x1, y1, x2, y2, w_grid, h_grid):
    cw = x2 - x1
    ch = y2 - y1
    a1 = cw * ch
    best_iou = jnp.zeros_like(cw)
    best_ind = jnp.zeros_like(cw)
    for a in range(_A):
        aw = float(_ANCHORS[a, 0])
        ah = float(_ANCHORS[a, 1])
        a2 = float(np.float32(_ANCHORS[a, 0]) * np.float32(_ANCHORS[a, 1]))
        inter = jnp.minimum(cw, aw) * jnp.minimum(ch, ah)
        union = jnp.clip(a1 + a2 - inter, 1e-12, None)
        iou = inter / union
        upd = iou > best_iou
        best_ind = jnp.where(upd, float(a), best_ind)
        best_iou = jnp.where(upd, iou, best_iou)
    dx = (x1 + x2) / 2.0 * w_grid
    dy = (y1 + y2) / 2.0 * h_grid
    gxk = jnp.ceil(dx) - 1.0
    gyk = jnp.ceil(dy) - 1.0
    pos = gyk * w_grid + gxk
    return pos, best_ind, best_iou, cw, ch, dx - gxk, dy - gyk


def _meta_body(t_ref, out_ref, *, grid_h, grid_w, t_ch, c_ch, bpp):
    eye5 = (lax.broadcasted_iota(jnp.int32, (5, 5), 0) ==
            lax.broadcasted_iota(jnp.int32, (5, 5), 1)).astype(jnp.float32)
    for bi in range(bpp):
        tb = t_ref[bi]
        tt = lax.dot_general(eye5, tb, (((1,), (1,)), ((), ())),
                             precision=_HP)
        pos, ind, miou, cw, ch, fx, fy = _per_truth(
            tt[0:1, :], tt[1:2, :], tt[2:3, :], tt[3:4, :],
            float(grid_w), float(grid_h))
        valid = miou != 0.0
        posj, indj, miouj = _per_truth(
            tb[:, 0:1], tb[:, 1:2], tb[:, 2:3], tb[:, 3:4],
            float(grid_w), float(grid_h))[:3]
        iota_j = lax.broadcasted_iota(jnp.int32, (_K, 1), 0)
        iota_m = lax.broadcasted_iota(jnp.int32, (1, _K), 1)
        beats = (posj == pos) & (indj == ind) & (
            (miouj > miou) | ((miouj == miou) & (iota_j < iota_m)))
        winner = valid & jnp.logical_not(jnp.any(beats, axis=0, keepdims=True))

        rdx = -jnp.log(1.0 / fx - 1.0)
        rdy = -jnp.log(1.0 / fy - 1.0)
        aw_sel = jnp.zeros_like(ind)
        ah_sel = jnp.zeros_like(ind)
        for a in range(_A):
            hit = ind == float(a)
            aw_sel = jnp.where(hit, float(_ANCHORS[a, 0]), aw_sel)
            ah_sel = jnp.where(hit, float(_ANCHORS[a, 1]), ah_sel)
        t2t = jnp.log(cw) / aw_sel
        t3t = jnp.log(ch) / ah_sel
        cf2 = 2.0 - cw * ch
        fb = pos * float(t_ch) + ind * float(c_ch)

        zero = jnp.zeros_like(fb)
        rows = [jnp.where(winner, fb, zero), jnp.where(winner, cf2, zero),
                jnp.where(winner, rdx, zero), jnp.where(winner, rdy, zero),
                jnp.where(winner, t2t, zero), jnp.where(winner, t3t, zero),
                zero, zero]
        pad = jnp.zeros((8, _KP - _K), jnp.float32)
        out_ref[bi] = jnp.concatenate(
            [jnp.concatenate(rows, axis=0), pad], axis=1)


def _make_sc_coord(b_total, elems_per_batch):
    info = plsc.get_sparse_core_info()
    nw = info.num_cores * info.num_subcores
    b_per_w = b_total // nw
    n_rows = 4 * b_per_w
    mesh = plsc.VectorSubcoreMesh(core_axis_name="c", subcore_axis_name="s")

    @functools.partial(
        pl.kernel, mesh=mesh,
        out_type=jax.ShapeDtypeStruct((b_total, 16), jnp.float32),
        scratch_types=[
            pltpu.VMEM((b_per_w, 8, _KP), jnp.float32),
            pltpu.VMEM((n_rows, _KP), jnp.int32),
            pltpu.VMEM((n_rows, _KP), jnp.float32),
            pltpu.VMEM((16,), jnp.float32),
            pltpu.SemaphoreType.DMA,
        ],
    )
    def sc_coord(xflat_hbm, meta_hbm, out_hbm, meta_v, idx_v, vals_v,
                 acc_v, sem):
        wid = lax.axis_index("s") * info.num_cores + lax.axis_index("c")
        base = wid * b_per_w
        for bi in range(b_per_w):
            pltpu.sync_copy(meta_hbm.at[base + bi], meta_v.at[bi])
        for bi in range(b_per_w):
            bb = (base + bi) * elems_per_batch
            for c in range(4):
                for chk in range(_KP // 16):
                    sl = pl.ds(chk * 16, 16)
                    fb = meta_v[bi, 0, sl]
                    idx_v[bi * 4 + c, sl] = fb.astype(jnp.int32) + (bb + c)
        cps = [pltpu.async_copy(xflat_hbm.at[idx_v.at[r]], vals_v.at[r], sem)
               for r in range(n_rows)]
        for cp in cps:
            cp.wait()
        for bi in range(b_per_w):
            acc = jnp.zeros((16,), jnp.float32)
            for chk in range(_KP // 16):
                sl = pl.ds(chk * 16, 16)
                wcf2 = meta_v[bi, 1, sl]
                d0 = vals_v[bi * 4 + 0, sl] - meta_v[bi, 2, sl]
                d1 = vals_v[bi * 4 + 1, sl] - meta_v[bi, 3, sl]
                d2 = vals_v[bi * 4 + 2, sl] - meta_v[bi, 4, sl]
                d3 = vals_v[bi * 4 + 3, sl] - meta_v[bi, 5, sl]
                acc = acc + wcf2 * (d0 * d0 + d1 * d1 + d2 * d2 + d3 * d3)
            acc_v[...] = acc
            pltpu.sync_copy(acc_v, out_hbm.at[base + bi])

    return sc_coord


def _dense_body(x_ref, tj_ref, out_ref, *, grid_h, grid_w, t_ch):
    hw = grid_h * grid_w
    xraw = x_ref[0]
    tj = tj_ref[0]

    c_ch = t_ch // _A
    r = lax.broadcasted_iota(jnp.int32, (_A * 5, t_ch), 0)
    t = lax.broadcasted_iota(jnp.int32, (_A * 5, t_ch), 1)
    sel = (t == ((r // 5) * c_ch + r % 5)).astype(jnp.float32)
    x = lax.dot_general(sel, xraw, (((1,), (1,)), ((), ())), precision=_HP)

    posj, indj, miouj = _per_truth(
        tj[:, 0:1], tj[:, 1:2], tj[:, 2:3], tj[:, 3:4],
        float(grid_w), float(grid_h))[:3]
    validj = miouj != 0.0

    cell = lax.broadcasted_iota(jnp.int32, (1, hw), 1)
    gx = (cell % grid_w).astype(jnp.float32)
    gy = (cell // grid_w).astype(jnp.float32)
    cellf = cell.astype(jnp.float32)

    tx1 = tj[:, 0:1]
    ty1 = tj[:, 1:2]
    tx2 = tj[:, 2:3]
    ty2 = tj[:, 3:4]
    a2t = 0.375 * ((tx2 - tx1) * (ty2 - ty1))

    acc_prior = jnp.float32(0.0)
    acc_noobj = jnp.float32(0.0)
    for a in range(_A):
        base = a * 5
        t0 = x[base + 0:base + 1, :]
        t1 = x[base + 1:base + 2, :]
        t2 = x[base + 2:base + 3, :]
        t3 = x[base + 3:base + 4, :]
        t4 = x[base + 4:base + 5, :]
        aw = float(_ANCHORS[a, 0])
        ah = float(_ANCHORS[a, 1])
        c0 = (1.0 / (1.0 + jnp.exp(-t0)) + gx) / float(grid_w)
        c1 = (1.0 / (1.0 + jnp.exp(-t1)) + gy) / float(grid_h)
        wa = jnp.exp(t2) * aw
        ha = jnp.exp(t3) * ah
        bx1 = c0 - wa / 2.0
        bx2 = c0 + wa / 2.0
        by1 = c1 - ha / 2.0
        by2 = c1 + ha / 2.0
        a1 = (bx2 - bx1) * (by2 - by1)
        acc_prior += jnp.sum((wa - aw) ** 2) + jnp.sum((ha - ah) ** 2)
        ix = jnp.clip(jnp.minimum(bx2, tx2) - jnp.maximum(bx1, tx1), 0.0, None)
        iy = jnp.clip(jnp.minimum(by2, ty2) - jnp.maximum(by1, ty1), 0.0, None)
        inter = ix * iy
        noobj = jnp.all(inter < (0.375 * a1 + a2t), axis=0, keepdims=True)
        match = (posj == cellf) & (indj == float(a)) & validj
        anymatch = jnp.any(match, axis=0, keepdims=True)
        acc_noobj += jnp.sum(
            jnp.where(noobj & jnp.logical_not(anymatch), t4, 0.0) ** 2)

    out_ref[0] = jnp.concatenate(
        [acc_prior.reshape(1, 1), acc_noobj.reshape(1, 1)], axis=1)


def kernel(output, truths, iteration):
    b, grid_h, grid_w, t = output.shape
    hw = grid_h * grid_w
    c_ch = t // _A
    k = truths.shape[1]
    bpp = 8

    meta = pl.pallas_call(
        functools.partial(_meta_body, grid_h=grid_h, grid_w=grid_w,
                          t_ch=t, c_ch=c_ch, bpp=bpp),
        grid=(b // bpp,),
        in_specs=[pl.BlockSpec((bpp, k, 5), lambda i: (i, 0, 0))],
        out_specs=pl.BlockSpec((bpp, 8, _KP), lambda i: (i, 0, 0)),
        out_shape=jax.ShapeDtypeStruct((b, 8, _KP), jnp.float32),
    )(truths)

    xflat = output.reshape(-1)
    coord_parts = _make_sc_coord(b, hw * t)(xflat, meta)

    x3 = output.reshape(b, hw, t)
    parts = pl.pallas_call(
        functools.partial(_dense_body, grid_h=grid_h, grid_w=grid_w, t_ch=t),
        grid=(b,),
        in_specs=[
            pl.BlockSpec((1, hw, t), lambda i: (i, 0, 0)),
            pl.BlockSpec((1, k, 5), lambda i: (i, 0, 0)),
        ],
        out_specs=pl.BlockSpec((1, 1, 2), lambda i: (i, 0, 0)),
        out_shape=jax.ShapeDtypeStruct((b, 1, 2), jnp.float32),
    )(x3, truths)

    sums = jnp.sum(parts, axis=(0, 1))
    prior = jnp.where(iteration < _PRIOR_ITER, sums[0], jnp.float32(0.0))
    return prior + sums[1] + jnp.sum(coord_parts)


# speedup vs baseline: 1.2802x; 1.0865x over previous
import functools

import jax
import jax.numpy as jnp
import numpy as np
from jax import lax
from jax.experimental import pallas as pl
from jax.experimental.pallas import tpu as pltpu
from jax.experimental.pallas import tpu_sc as plsc

_ANCHORS = np.array(
    [[1.3221 / 13.0, 1.73145 / 13.0],
     [3.19275 / 13.0, 4.00944 / 13.0],
     [5.05587 / 13.0, 8.09892 / 13.0],
     [9.47112 / 13.0, 4.84053 / 13.0],
     [11.2364 / 13.0, 10.0071 / 13.0]], dtype=np.float32)
_THRESH = 0.6
_PRIOR_ITER = 12800

_A = 5
_K = 50
_KP = 64
_HP = jax.lax.Precision.HIGHEST


def _per_truth(x1, y1, x2, y2, w_grid, h_grid):
    cw = x2 - x1
    ch = y2 - y1
    a1 = cw * ch
    best_iou = jnp.zeros_like(cw)
    best_ind = jnp.zeros_like(cw)
    for a in range(_A):
        aw = float(_ANCHORS[a, 0])
        ah = float(_ANCHORS[a, 1])
        a2 = float(np.float32(_ANCHORS[a, 0]) * np.float32(_ANCHORS[a, 1]))
        inter = jnp.minimum(cw, aw) * jnp.minimum(ch, ah)
        union = jnp.clip(a1 + a2 - inter, 1e-12, None)
        iou = inter / union
        upd = iou > best_iou
        best_ind = jnp.where(upd, float(a), best_ind)
        best_iou = jnp.where(upd, iou, best_iou)
    dx = (x1 + x2) / 2.0 * w_grid
    dy = (y1 + y2) / 2.0 * h_grid
    gxk = jnp.ceil(dx) - 1.0
    gyk = jnp.ceil(dy) - 1.0
    pos = gyk * w_grid + gxk
    return pos, best_ind, best_iou, cw, ch, dx - gxk, dy - gyk


def _meta_body(t_ref, out_ref, *, grid_h, grid_w, t_ch, c_ch, bpp):
    eye5 = (lax.broadcasted_iota(jnp.int32, (5, 5), 0) ==
            lax.broadcasted_iota(jnp.int32, (5, 5), 1)).astype(jnp.float32)
    for bi in range(bpp):
        tb = t_ref[bi]
        tt = lax.dot_general(eye5, tb, (((1,), (1,)), ((), ())),
                             precision=_HP)
        pos, ind, miou, cw, ch, fx, fy = _per_truth(
            tt[0:1, :], tt[1:2, :], tt[2:3, :], tt[3:4, :],
            float(grid_w), float(grid_h))
        valid = miou != 0.0
        posj, indj, miouj = _per_truth(
            tb[:, 0:1], tb[:, 1:2], tb[:, 2:3], tb[:, 3:4],
            float(grid_w), float(grid_h))[:3]
        iota_j = lax.broadcasted_iota(jnp.int32, (_K, 1), 0)
        iota_m = lax.broadcasted_iota(jnp.int32, (1, _K), 1)
        beats = (posj == pos) & (indj == ind) & (
            (miouj > miou) | ((miouj == miou) & (iota_j < iota_m)))
        winner = valid & jnp.logical_not(jnp.any(beats, axis=0, keepdims=True))

        rdx = -jnp.log(1.0 / fx - 1.0)
        rdy = -jnp.log(1.0 / fy - 1.0)
        aw_sel = jnp.zeros_like(ind)
        ah_sel = jnp.zeros_like(ind)
        for a in range(_A):
            hit = ind == float(a)
            aw_sel = jnp.where(hit, float(_ANCHORS[a, 0]), aw_sel)
            ah_sel = jnp.where(hit, float(_ANCHORS[a, 1]), ah_sel)
        t2t = jnp.log(cw) / aw_sel
        t3t = jnp.log(ch) / ah_sel
        cf2 = 2.0 - cw * ch
        fb = pos * float(t_ch) + ind * float(c_ch)

        zero = jnp.zeros_like(fb)
        rows = [jnp.where(winner, fb, zero), jnp.where(winner, cf2, zero),
                jnp.where(winner, rdx, zero), jnp.where(winner, rdy, zero),
                jnp.where(winner, t2t, zero), jnp.where(winner, t3t, zero),
                zero, zero]
        pad = jnp.zeros((8, _KP - _K), jnp.float32)
        out_ref[bi] = jnp.concatenate(
            [jnp.concatenate(rows, axis=0), pad], axis=1)


def _make_sc_coord(b_total, elems_per_batch):
    info = plsc.get_sparse_core_info()
    nw = info.num_cores * info.num_subcores
    b_per_w = b_total // nw
    n_rows = 4 * b_per_w
    mesh = plsc.VectorSubcoreMesh(core_axis_name="c", subcore_axis_name="s")

    @functools.partial(
        pl.kernel, mesh=mesh,
        out_type=jax.ShapeDtypeStruct((b_total, 16), jnp.float32),
        scratch_types=[
            pltpu.VMEM((b_per_w, 8, _KP), jnp.float32),
            pltpu.VMEM((n_rows, _KP), jnp.int32),
            pltpu.VMEM((n_rows, _KP), jnp.float32),
            pltpu.VMEM((16,), jnp.float32),
            pltpu.SemaphoreType.DMA,
        ],
    )
    def sc_coord(xflat_hbm, meta_hbm, out_hbm, meta_v, idx_v, vals_v,
                 acc_v, sem):
        wid = lax.axis_index("s") * info.num_cores + lax.axis_index("c")
        base = wid * b_per_w
        for bi in range(b_per_w):
            pltpu.sync_copy(meta_hbm.at[base + bi], meta_v.at[bi])
        for bi in range(b_per_w):
            bb = (base + bi) * elems_per_batch
            for c in range(4):
                for chk in range(_KP // 16):
                    sl = pl.ds(chk * 16, 16)
                    fb = meta_v[bi, 0, sl]
                    idx_v[bi * 4 + c, sl] = fb.astype(jnp.int32) + (bb + c)
        cps = [pltpu.async_copy(xflat_hbm.at[idx_v.at[r]], vals_v.at[r], sem)
               for r in range(n_rows)]
        for cp in cps:
            cp.wait()
        for bi in range(b_per_w):
            acc = jnp.zeros((16,), jnp.float32)
            for chk in range(_KP // 16):
                sl = pl.ds(chk * 16, 16)
                wcf2 = meta_v[bi, 1, sl]
                d0 = vals_v[bi * 4 + 0, sl] - meta_v[bi, 2, sl]
                d1 = vals_v[bi * 4 + 1, sl] - meta_v[bi, 3, sl]
                d2 = vals_v[bi * 4 + 2, sl] - meta_v[bi, 4, sl]
                d3 = vals_v[bi * 4 + 3, sl] - meta_v[bi, 5, sl]
                acc = acc + wcf2 * (d0 * d0 + d1 * d1 + d2 * d2 + d3 * d3)
            acc_v[...] = acc
            pltpu.sync_copy(acc_v, out_hbm.at[base + bi])

    return sc_coord


def _dense_body(x_ref, tj_ref, out_ref, *, grid_h, grid_w, t_ch, bpp):
    hw = grid_h * grid_w
    c_ch = t_ch // _A
    r = lax.broadcasted_iota(jnp.int32, (_A * 5, t_ch), 0)
    t = lax.broadcasted_iota(jnp.int32, (_A * 5, t_ch), 1)
    sel = (t == ((r // 5) * c_ch + r % 5)).astype(jnp.float32)

    cell = lax.broadcasted_iota(jnp.int32, (1, hw), 1)
    gx = (cell % grid_w).astype(jnp.float32)
    gy = (cell // grid_w).astype(jnp.float32)
    cellf = cell.astype(jnp.float32)

    for bi in range(bpp):
        xraw = x_ref[bi]
        tj = tj_ref[bi]
        x = lax.dot_general(sel, xraw, (((1,), (1,)), ((), ())), precision=_HP)

        posj, indj, miouj = _per_truth(
            tj[:, 0:1], tj[:, 1:2], tj[:, 2:3], tj[:, 3:4],
            float(grid_w), float(grid_h))[:3]
        validj = miouj != 0.0

        tx1 = tj[:, 0:1]
        ty1 = tj[:, 1:2]
        tx2 = tj[:, 2:3]
        ty2 = tj[:, 3:4]
        a2t = 0.375 * ((tx2 - tx1) * (ty2 - ty1))

        acc_prior = jnp.float32(0.0)
        acc_noobj = jnp.float32(0.0)
        for a in range(_A):
            base = a * 5
            t0 = x[base + 0:base + 1, :]
            t1 = x[base + 1:base + 2, :]
            t2 = x[base + 2:base + 3, :]
            t3 = x[base + 3:base + 4, :]
            t4 = x[base + 4:base + 5, :]
            aw = float(_ANCHORS[a, 0])
            ah = float(_ANCHORS[a, 1])
            c0 = (1.0 / (1.0 + jnp.exp(-t0)) + gx) / float(grid_w)
            c1 = (1.0 / (1.0 + jnp.exp(-t1)) + gy) / float(grid_h)
            wa = jnp.exp(t2) * aw
            ha = jnp.exp(t3) * ah
            bx1 = c0 - wa / 2.0
            bx2 = c0 + wa / 2.0
            by1 = c1 - ha / 2.0
            by2 = c1 + ha / 2.0
            a1 = (bx2 - bx1) * (by2 - by1)
            acc_prior += jnp.sum((wa - aw) ** 2) + jnp.sum((ha - ah) ** 2)
            ix = jnp.clip(jnp.minimum(bx2, tx2) - jnp.maximum(bx1, tx1),
                          0.0, None)
            iy = jnp.clip(jnp.minimum(by2, ty2) - jnp.maximum(by1, ty1),
                          0.0, None)
            inter = ix * iy
            noobj = jnp.all(inter < (0.375 * a1 + a2t), axis=0, keepdims=True)
            match = (posj == cellf) & (indj == float(a)) & validj
            anymatch = jnp.any(match, axis=0, keepdims=True)
            acc_noobj += jnp.sum(
                jnp.where(noobj & jnp.logical_not(anymatch), t4, 0.0) ** 2)

        out_ref[bi] = jnp.concatenate(
            [acc_prior.reshape(1, 1), acc_noobj.reshape(1, 1)], axis=1)


def kernel(output, truths, iteration):
    b, grid_h, grid_w, t = output.shape
    hw = grid_h * grid_w
    c_ch = t // _A
    k = truths.shape[1]
    bpp = 8

    meta = pl.pallas_call(
        functools.partial(_meta_body, grid_h=grid_h, grid_w=grid_w,
                          t_ch=t, c_ch=c_ch, bpp=bpp),
        grid=(b // bpp,),
        in_specs=[pl.BlockSpec((bpp, k, 5), lambda i: (i, 0, 0))],
        out_specs=pl.BlockSpec((bpp, 8, _KP), lambda i: (i, 0, 0)),
        out_shape=jax.ShapeDtypeStruct((b, 8, _KP), jnp.float32),
    )(truths)

    xflat = output.reshape(-1)
    coord_parts = _make_sc_coord(b, hw * t)(xflat, meta)

    dpp = 4
    x3 = output.reshape(b, hw, t)
    parts = pl.pallas_call(
        functools.partial(_dense_body, grid_h=grid_h, grid_w=grid_w, t_ch=t,
                          bpp=dpp),
        grid=(b // dpp,),
        in_specs=[
            pl.BlockSpec((dpp, hw, t), lambda i: (i, 0, 0)),
            pl.BlockSpec((dpp, k, 5), lambda i: (i, 0, 0)),
        ],
        out_specs=pl.BlockSpec((dpp, 1, 2), lambda i: (i, 0, 0)),
        out_shape=jax.ShapeDtypeStruct((b, 1, 2), jnp.float32),
    )(x3, truths)

    sums = jnp.sum(parts, axis=(0, 1))
    prior = jnp.where(iteration < _PRIOR_ITER, sums[0], jnp.float32(0.0))
    return prior + sums[1] + jnp.sum(coord_parts)


# speedup vs baseline: 1.5088x; 1.1786x over previous
import functools

import jax
import jax.numpy as jnp
import numpy as np
from jax import lax
from jax.experimental import pallas as pl
from jax.experimental.pallas import tpu as pltpu
from jax.experimental.pallas import tpu_sc as plsc

_ANCHORS = np.array(
    [[1.3221 / 13.0, 1.73145 / 13.0],
     [3.19275 / 13.0, 4.00944 / 13.0],
     [5.05587 / 13.0, 8.09892 / 13.0],
     [9.47112 / 13.0, 4.84053 / 13.0],
     [11.2364 / 13.0, 10.0071 / 13.0]], dtype=np.float32)
_THRESH = 0.6
_PRIOR_ITER = 12800

_A = 5
_K = 50
_KP = 64
_HP = jax.lax.Precision.HIGHEST


def _per_truth(x1, y1, x2, y2, w_grid, h_grid):
    cw = x2 - x1
    ch = y2 - y1
    a1 = cw * ch
    best_iou = jnp.zeros_like(cw)
    best_ind = jnp.zeros_like(cw)
    for a in range(_A):
        aw = float(_ANCHORS[a, 0])
        ah = float(_ANCHORS[a, 1])
        a2 = float(np.float32(_ANCHORS[a, 0]) * np.float32(_ANCHORS[a, 1]))
        inter = jnp.minimum(cw, aw) * jnp.minimum(ch, ah)
        union = jnp.clip(a1 + a2 - inter, 1e-12, None)
        iou = inter / union
        upd = iou > best_iou
        best_ind = jnp.where(upd, float(a), best_ind)
        best_iou = jnp.where(upd, iou, best_iou)
    dx = (x1 + x2) / 2.0 * w_grid
    dy = (y1 + y2) / 2.0 * h_grid
    gxk = jnp.ceil(dx) - 1.0
    gyk = jnp.ceil(dy) - 1.0
    pos = gyk * w_grid + gxk
    return pos, best_ind, best_iou, cw, ch, dx - gxk, dy - gyk


def _prep_body(x_ref, t_ref, xt_ref, meta_ref, *, grid_h, grid_w, t_ch, bpp):
    hw = grid_h * grid_w
    c_ch = t_ch // _A
    r = lax.broadcasted_iota(jnp.int32, (_A * 5, t_ch), 0)
    t = lax.broadcasted_iota(jnp.int32, (_A * 5, t_ch), 1)
    sel = (t == ((r // 5) * c_ch + r % 5)).astype(jnp.float32)
    eye5 = (lax.broadcasted_iota(jnp.int32, (5, 5), 0) ==
            lax.broadcasted_iota(jnp.int32, (5, 5), 1)).astype(jnp.float32)

    for bi in range(bpp):
        xraw = x_ref[bi]
        xt_ref[bi] = lax.dot_general(sel, xraw, (((1,), (1,)), ((), ())),
                                     precision=_HP)
        tb = t_ref[bi]
        tt = lax.dot_general(eye5, tb, (((1,), (1,)), ((), ())),
                             precision=_HP)
        pos, ind, miou, cw, ch, fx, fy = _per_truth(
            tt[0:1, :], tt[1:2, :], tt[2:3, :], tt[3:4, :],
            float(grid_w), float(grid_h))
        valid = miou != 0.0
        posj, indj, miouj = _per_truth(
            tb[:, 0:1], tb[:, 1:2], tb[:, 2:3], tb[:, 3:4],
            float(grid_w), float(grid_h))[:3]
        iota_j = lax.broadcasted_iota(jnp.int32, (_K, 1), 0)
        iota_m = lax.broadcasted_iota(jnp.int32, (1, _K), 1)
        beats = (posj == pos) & (indj == ind) & (
            (miouj > miou) | ((miouj == miou) & (iota_j < iota_m)))
        winner = valid & jnp.logical_not(jnp.any(beats, axis=0, keepdims=True))

        rdx = -jnp.log(1.0 / fx - 1.0)
        rdy = -jnp.log(1.0 / fy - 1.0)
        aw_sel = jnp.zeros_like(ind)
        ah_sel = jnp.zeros_like(ind)
        for a in range(_A):
            hit = ind == float(a)
            aw_sel = jnp.where(hit, float(_ANCHORS[a, 0]), aw_sel)
            ah_sel = jnp.where(hit, float(_ANCHORS[a, 1]), ah_sel)
        t2t = jnp.log(cw) / aw_sel
        t3t = jnp.log(ch) / ah_sel
        cf2 = 2.0 - cw * ch
        fb = pos + ind * float(5 * hw)

        zero = jnp.zeros_like(fb)
        rows = [jnp.where(winner, fb, zero), jnp.where(winner, cf2, zero),
                jnp.where(winner, rdx, zero), jnp.where(winner, rdy, zero),
                jnp.where(winner, t2t, zero), jnp.where(winner, t3t, zero),
                zero, zero]
        pad = jnp.zeros((8, _KP - _K), jnp.float32)
        meta_ref[bi] = jnp.concatenate(
            [jnp.concatenate(rows, axis=0), pad], axis=1)


def _make_sc_coord(b_total, elems_per_batch, ch_stride):
    info = plsc.get_sparse_core_info()
    nw = info.num_cores * info.num_subcores
    b_per_w = b_total // nw
    n_rows = 4 * b_per_w
    mesh = plsc.VectorSubcoreMesh(core_axis_name="c", subcore_axis_name="s")

    @functools.partial(
        pl.kernel, mesh=mesh,
        out_type=jax.ShapeDtypeStruct((b_total, 16), jnp.float32),
        scratch_types=[
            pltpu.VMEM((b_per_w, 8, _KP), jnp.float32),
            pltpu.VMEM((n_rows, _KP), jnp.int32),
            pltpu.VMEM((n_rows, _KP), jnp.float32),
            pltpu.VMEM((16,), jnp.float32),
            pltpu.SemaphoreType.DMA,
        ],
    )
    def sc_coord(xflat_hbm, meta_hbm, out_hbm, meta_v, idx_v, vals_v,
                 acc_v, sem):
        wid = lax.axis_index("s") * info.num_cores + lax.axis_index("c")
        base = wid * b_per_w
        for bi in range(b_per_w):
            pltpu.sync_copy(meta_hbm.at[base + bi], meta_v.at[bi])
        for bi in range(b_per_w):
            bb = (base + bi) * elems_per_batch
            for c in range(4):
                for chk in range(_KP // 16):
                    sl = pl.ds(chk * 16, 16)
                    fb = meta_v[bi, 0, sl]
                    idx_v[bi * 4 + c, sl] = (
                        fb.astype(jnp.int32) + (bb + c * ch_stride))
        cps = [pltpu.async_copy(xflat_hbm.at[idx_v.at[r]], vals_v.at[r], sem)
               for r in range(n_rows)]
        for cp in cps:
            cp.wait()
        for bi in range(b_per_w):
            acc = jnp.zeros((16,), jnp.float32)
            for chk in range(_KP // 16):
                sl = pl.ds(chk * 16, 16)
                wcf2 = meta_v[bi, 1, sl]
                d0 = vals_v[bi * 4 + 0, sl] - meta_v[bi, 2, sl]
                d1 = vals_v[bi * 4 + 1, sl] - meta_v[bi, 3, sl]
                d2 = vals_v[bi * 4 + 2, sl] - meta_v[bi, 4, sl]
                d3 = vals_v[bi * 4 + 3, sl] - meta_v[bi, 5, sl]
                acc = acc + wcf2 * (d0 * d0 + d1 * d1 + d2 * d2 + d3 * d3)
            acc_v[...] = acc
            pltpu.sync_copy(acc_v, out_hbm.at[base + bi])

    return sc_coord


def _dense_body(x_ref, tj_ref, out_ref, *, grid_h, grid_w, bpp):
    hw = grid_h * grid_w
    cell = lax.broadcasted_iota(jnp.int32, (1, hw), 1)
    gx = (cell % grid_w).astype(jnp.float32)
    gy = (cell // grid_w).astype(jnp.float32)
    cellf = cell.astype(jnp.float32)
    bf = jnp.bfloat16

    for bi in range(bpp):
        x = x_ref[bi]
        tj = tj_ref[bi]

        posj, indj, miouj = _per_truth(
            tj[:, 0:1], tj[:, 1:2], tj[:, 2:3], tj[:, 3:4],
            float(grid_w), float(grid_h))[:3]
        validj = miouj != 0.0

        tx1 = tj[:, 0:1].astype(bf)
        ty1 = tj[:, 1:2].astype(bf)
        tx2 = tj[:, 2:3].astype(bf)
        ty2 = tj[:, 3:4].astype(bf)
        a2t = (0.375 * ((tj[:, 2:3] - tj[:, 0:1]) *
                        (tj[:, 3:4] - tj[:, 1:2]))).astype(bf)

        acc_prior = jnp.float32(0.0)
        acc_noobj = jnp.float32(0.0)
        for a in range(_A):
            base = a * 5
            t0 = x[base + 0:base + 1, :]
            t1 = x[base + 1:base + 2, :]
            t2 = x[base + 2:base + 3, :]
            t3 = x[base + 3:base + 4, :]
            t4 = x[base + 4:base + 5, :]
            aw = float(_ANCHORS[a, 0])
            ah = float(_ANCHORS[a, 1])
            c0 = (1.0 / (1.0 + jnp.exp(-t0)) + gx) / float(grid_w)
            c1 = (1.0 / (1.0 + jnp.exp(-t1)) + gy) / float(grid_h)
            wa = jnp.exp(t2) * aw
            ha = jnp.exp(t3) * ah
            bx1 = c0 - wa / 2.0
            bx2 = c0 + wa / 2.0
            by1 = c1 - ha / 2.0
            by2 = c1 + ha / 2.0
            a1 = (bx2 - bx1) * (by2 - by1)
            acc_prior += jnp.sum((wa - aw) ** 2) + jnp.sum((ha - ah) ** 2)
            ixh = jnp.clip(jnp.minimum(bx2.astype(bf), tx2) -
                           jnp.maximum(bx1.astype(bf), tx1), 0.0, None)
            iyh = jnp.clip(jnp.minimum(by2.astype(bf), ty2) -
                           jnp.maximum(by1.astype(bf), ty1), 0.0, None)
            inter = ixh * iyh
            thr = (0.375 * a1).astype(bf) + a2t
            viol = jnp.max(inter - thr, axis=0, keepdims=True)
            noobj = viol.astype(jnp.float32) < 0.0
            match = (posj == cellf) & (indj == float(a)) & validj
            anymatch = jnp.any(match, axis=0, keepdims=True)
            acc_noobj += jnp.sum(
                jnp.where(noobj & jnp.logical_not(anymatch), t4, 0.0) ** 2)

        out_ref[bi] = jnp.concatenate(
            [acc_prior.reshape(1, 1), acc_noobj.reshape(1, 1)], axis=1)


def kernel(output, truths, iteration):
    b, grid_h, grid_w, t = output.shape
    hw = grid_h * grid_w
    k = truths.shape[1]
    ppp = 8
    dpp = 8

    x3 = output.reshape(b, hw, t)
    xt, meta = pl.pallas_call(
        functools.partial(_prep_body, grid_h=grid_h, grid_w=grid_w,
                          t_ch=t, bpp=ppp),
        grid=(b // ppp,),
        in_specs=[
            pl.BlockSpec((ppp, hw, t), lambda i: (i, 0, 0)),
            pl.BlockSpec((ppp, k, 5), lambda i: (i, 0, 0)),
        ],
        out_specs=[
            pl.BlockSpec((ppp, _A * 5, hw), lambda i: (i, 0, 0)),
            pl.BlockSpec((ppp, 8, _KP), lambda i: (i, 0, 0)),
        ],
        out_shape=[
            jax.ShapeDtypeStruct((b, _A * 5, hw), jnp.float32),
            jax.ShapeDtypeStruct((b, 8, _KP), jnp.float32),
        ],
    )(x3, truths)

    xtflat = xt.reshape(-1)
    coord_parts = _make_sc_coord(b, _A * 5 * hw, hw)(xtflat, meta)

    parts = pl.pallas_call(
        functools.partial(_dense_body, grid_h=grid_h, grid_w=grid_w, bpp=dpp),
        grid=(b // dpp,),
        in_specs=[
            pl.BlockSpec((dpp, _A * 5, hw), lambda i: (i, 0, 0)),
            pl.BlockSpec((dpp, k, 5), lambda i: (i, 0, 0)),
        ],
        out_specs=pl.BlockSpec((dpp, 1, 2), lambda i: (i, 0, 0)),
        out_shape=jax.ShapeDtypeStruct((b, 1, 2), jnp.float32),
    )(xt, truths)

    sums = jnp.sum(parts, axis=(0, 1))
    prior = jnp.where(iteration < _PRIOR_ITER, sums[0], jnp.float32(0.0))
    return prior + sums[1] + jnp.sum(coord_parts)


# speedup vs baseline: 1.8757x; 1.2432x over previous
import functools

import jax
import jax.numpy as jnp
import numpy as np
from jax import lax
from jax.experimental import pallas as pl
from jax.experimental.pallas import tpu as pltpu
from jax.experimental.pallas import tpu_sc as plsc

_ANCHORS = np.array(
    [[1.3221 / 13.0, 1.73145 / 13.0],
     [3.19275 / 13.0, 4.00944 / 13.0],
     [5.05587 / 13.0, 8.09892 / 13.0],
     [9.47112 / 13.0, 4.84053 / 13.0],
     [11.2364 / 13.0, 10.0071 / 13.0]], dtype=np.float32)
_THRESH = 0.6
_PRIOR_ITER = 12800

_A = 5
_K = 50
_KP = 64
_HP = jax.lax.Precision.HIGHEST


def _per_truth(x1, y1, x2, y2, w_grid, h_grid):
    cw = x2 - x1
    ch = y2 - y1
    a1 = cw * ch
    best_iou = jnp.zeros_like(cw)
    best_ind = jnp.zeros_like(cw)
    for a in range(_A):
        aw = float(_ANCHORS[a, 0])
        ah = float(_ANCHORS[a, 1])
        a2 = float(np.float32(_ANCHORS[a, 0]) * np.float32(_ANCHORS[a, 1]))
        inter = jnp.minimum(cw, aw) * jnp.minimum(ch, ah)
        union = jnp.clip(a1 + a2 - inter, 1e-12, None)
        iou = inter / union
        upd = iou > best_iou
        best_ind = jnp.where(upd, float(a), best_ind)
        best_iou = jnp.where(upd, iou, best_iou)
    dx = (x1 + x2) / 2.0 * w_grid
    dy = (y1 + y2) / 2.0 * h_grid
    gxk = jnp.ceil(dx) - 1.0
    gyk = jnp.ceil(dy) - 1.0
    pos = gyk * w_grid + gxk
    return pos, best_ind, best_iou, cw, ch, dx - gxk, dy - gyk


def _prep_body(x_ref, t_ref, xt_ref, meta_ref, out_ref, *, grid_h, grid_w,
               t_ch, bpp):
    hw = grid_h * grid_w
    c_ch = t_ch // _A
    r = lax.broadcasted_iota(jnp.int32, (_A * 5, t_ch), 0)
    t = lax.broadcasted_iota(jnp.int32, (_A * 5, t_ch), 1)
    sel = (t == ((r // 5) * c_ch + r % 5)).astype(jnp.float32)
    eye5 = (lax.broadcasted_iota(jnp.int32, (5, 5), 0) ==
            lax.broadcasted_iota(jnp.int32, (5, 5), 1)).astype(jnp.float32)
    cell = lax.broadcasted_iota(jnp.int32, (1, hw), 1)
    gx = (cell % grid_w).astype(jnp.float32)
    gy = (cell // grid_w).astype(jnp.float32)
    cellf = cell.astype(jnp.float32)
    bf = jnp.bfloat16

    for bi in range(bpp):
        xraw = x_ref[bi]
        x = lax.dot_general(sel, xraw, (((1,), (1,)), ((), ())),
                            precision=_HP)
        xt_ref[bi] = x
        tb = t_ref[bi]
        tt = lax.dot_general(eye5, tb, (((1,), (1,)), ((), ())),
                             precision=_HP)
        pos, ind, miou, cw, ch, fx, fy = _per_truth(
            tt[0:1, :], tt[1:2, :], tt[2:3, :], tt[3:4, :],
            float(grid_w), float(grid_h))
        valid = miou != 0.0
        posj, indj, miouj = _per_truth(
            tb[:, 0:1], tb[:, 1:2], tb[:, 2:3], tb[:, 3:4],
            float(grid_w), float(grid_h))[:3]
        iota_j = lax.broadcasted_iota(jnp.int32, (_K, 1), 0)
        iota_m = lax.broadcasted_iota(jnp.int32, (1, _K), 1)
        beats = (posj == pos) & (indj == ind) & (
            (miouj > miou) | ((miouj == miou) & (iota_j < iota_m)))
        winner = valid & jnp.logical_not(jnp.any(beats, axis=0, keepdims=True))

        rdx = -jnp.log(1.0 / fx - 1.0)
        rdy = -jnp.log(1.0 / fy - 1.0)
        aw_sel = jnp.zeros_like(ind)
        ah_sel = jnp.zeros_like(ind)
        for a in range(_A):
            hit = ind == float(a)
            aw_sel = jnp.where(hit, float(_ANCHORS[a, 0]), aw_sel)
            ah_sel = jnp.where(hit, float(_ANCHORS[a, 1]), ah_sel)
        t2t = jnp.log(cw) / aw_sel
        t3t = jnp.log(ch) / ah_sel
        cf2 = 2.0 - cw * ch
        fb = pos + ind * float(5 * hw)

        zero = jnp.zeros_like(fb)
        rows = [jnp.where(winner, fb, zero), jnp.where(winner, cf2, zero),
                jnp.where(winner, rdx, zero), jnp.where(winner, rdy, zero),
                jnp.where(winner, t2t, zero), jnp.where(winner, t3t, zero),
                zero, zero]
        pad = jnp.zeros((8, _KP - _K), jnp.float32)
        meta_ref[bi] = jnp.concatenate(
            [jnp.concatenate(rows, axis=0), pad], axis=1)

        validj = miouj != 0.0
        tx1 = tb[:, 0:1].astype(bf)
        ty1 = tb[:, 1:2].astype(bf)
        tx2 = tb[:, 2:3].astype(bf)
        ty2 = tb[:, 3:4].astype(bf)
        a2t = (0.375 * ((tb[:, 2:3] - tb[:, 0:1]) *
                        (tb[:, 3:4] - tb[:, 1:2]))).astype(bf)
        acc_prior = jnp.float32(0.0)
        acc_noobj = jnp.float32(0.0)
        for a in range(_A):
            base = a * 5
            t0 = x[base + 0:base + 1, :]
            t1 = x[base + 1:base + 2, :]
            t2 = x[base + 2:base + 3, :]
            t3 = x[base + 3:base + 4, :]
            t4 = x[base + 4:base + 5, :]
            aw = float(_ANCHORS[a, 0])
            ah = float(_ANCHORS[a, 1])
            c0 = (1.0 / (1.0 + jnp.exp(-t0)) + gx) / float(grid_w)
            c1 = (1.0 / (1.0 + jnp.exp(-t1)) + gy) / float(grid_h)
            wa = jnp.exp(t2) * aw
            ha = jnp.exp(t3) * ah
            bx1 = c0 - wa / 2.0
            bx2 = c0 + wa / 2.0
            by1 = c1 - ha / 2.0
            by2 = c1 + ha / 2.0
            a1 = (bx2 - bx1) * (by2 - by1)
            acc_prior += jnp.sum((wa - aw) ** 2) + jnp.sum((ha - ah) ** 2)
            ixh = jnp.clip(jnp.minimum(bx2.astype(bf), tx2) -
                           jnp.maximum(bx1.astype(bf), tx1), 0.0, None)
            iyh = jnp.clip(jnp.minimum(by2.astype(bf), ty2) -
                           jnp.maximum(by1.astype(bf), ty1), 0.0, None)
            inter = ixh * iyh
            thr = (0.375 * a1).astype(bf) + a2t
            viol = jnp.max(inter - thr, axis=0, keepdims=True)
            noobj = viol.astype(jnp.float32) < 0.0
            match = (posj == cellf) & (indj == float(a)) & validj
            anymatch = jnp.any(match, axis=0, keepdims=True)
            acc_noobj += jnp.sum(
                jnp.where(noobj & jnp.logical_not(anymatch), t4, 0.0) ** 2)

        out_ref[bi] = jnp.concatenate(
            [acc_prior.reshape(1, 1), acc_noobj.reshape(1, 1)], axis=1)


def _make_sc_coord(b_total, elems_per_batch, ch_stride):
    info = plsc.get_sparse_core_info()
    nw = info.num_cores * info.num_subcores
    b_per_w = b_total // nw
    n_rows = 4 * b_per_w
    mesh = plsc.VectorSubcoreMesh(core_axis_name="c", subcore_axis_name="s")

    @functools.partial(
        pl.kernel, mesh=mesh,
        out_type=jax.ShapeDtypeStruct((b_total, 16), jnp.float32),
        scratch_types=[
            pltpu.VMEM((b_per_w, 8, _KP), jnp.float32),
            pltpu.VMEM((n_rows, _KP), jnp.int32),
            pltpu.VMEM((n_rows, _KP), jnp.float32),
            pltpu.VMEM((16,), jnp.float32),
            pltpu.SemaphoreType.DMA,
        ],
    )
    def sc_coord(xflat_hbm, meta_hbm, out_hbm, meta_v, idx_v, vals_v,
                 acc_v, sem):
        wid = lax.axis_index("s") * info.num_cores + lax.axis_index("c")
        base = wid * b_per_w
        for bi in range(b_per_w):
            pltpu.sync_copy(meta_hbm.at[base + bi], meta_v.at[bi])
        for bi in range(b_per_w):
            bb = (base + bi) * elems_per_batch
            for c in range(4):
                for chk in range(_KP // 16):
                    sl = pl.ds(chk * 16, 16)
                    fb = meta_v[bi, 0, sl]
                    idx_v[bi * 4 + c, sl] = (
                        fb.astype(jnp.int32) + (bb + c * ch_stride))
        cps = [pltpu.async_copy(xflat_hbm.at[idx_v.at[r]], vals_v.at[r], sem)
               for r in range(n_rows)]
        for cp in cps:
            cp.wait()
        for bi in range(b_per_w):
            acc = jnp.zeros((16,), jnp.float32)
            for chk in range(_KP // 16):
                sl = pl.ds(chk * 16, 16)
                wcf2 = meta_v[bi, 1, sl]
                d0 = vals_v[bi * 4 + 0, sl] - meta_v[bi, 2, sl]
                d1 = vals_v[bi * 4 + 1, sl] - meta_v[bi, 3, sl]
                d2 = vals_v[bi * 4 + 2, sl] - meta_v[bi, 4, sl]
                d3 = vals_v[bi * 4 + 3, sl] - meta_v[bi, 5, sl]
                acc = acc + wcf2 * (d0 * d0 + d1 * d1 + d2 * d2 + d3 * d3)
            acc_v[...] = acc
            pltpu.sync_copy(acc_v, out_hbm.at[base + bi])

    return sc_coord


def kernel(output, truths, iteration):
    b, grid_h, grid_w, t = output.shape
    hw = grid_h * grid_w
    k = truths.shape[1]
    ppp = 8

    x3 = output.reshape(b, hw, t)
    xt, meta, parts = pl.pallas_call(
        functools.partial(_prep_body, grid_h=grid_h, grid_w=grid_w,
                          t_ch=t, bpp=ppp),
        grid=(b // ppp,),
        in_specs=[
            pl.BlockSpec((ppp, hw, t), lambda i: (i, 0, 0)),
            pl.BlockSpec((ppp, k, 5), lambda i: (i, 0, 0)),
        ],
        out_specs=[
            pl.BlockSpec((ppp, _A * 5, hw), lambda i: (i, 0, 0)),
            pl.BlockSpec((ppp, 8, _KP), lambda i: (i, 0, 0)),
            pl.BlockSpec((ppp, 1, 2), lambda i: (i, 0, 0)),
        ],
        out_shape=[
            jax.ShapeDtypeStruct((b, _A * 5, hw), jnp.float32),
            jax.ShapeDtypeStruct((b, 8, _KP), jnp.float32),
            jax.ShapeDtypeStruct((b, 1, 2), jnp.float32),
        ],
    )(x3, truths)

    xtflat = xt.reshape(-1)
    coord_parts = _make_sc_coord(b, _A * 5 * hw, hw)(xtflat, meta)

    sums = jnp.sum(parts, axis=(0, 1))
    prior = jnp.where(iteration < _PRIOR_ITER, sums[0], jnp.float32(0.0))
    return prior + sums[1] + jnp.sum(coord_parts)


# speedup vs baseline: 1.8762x; 1.0003x over previous
import functools

import jax
import jax.numpy as jnp
import numpy as np
from jax import lax
from jax.experimental import pallas as pl
from jax.experimental.pallas import tpu as pltpu
from jax.experimental.pallas import tpu_sc as plsc

_ANCHORS = np.array(
    [[1.3221 / 13.0, 1.73145 / 13.0],
     [3.19275 / 13.0, 4.00944 / 13.0],
     [5.05587 / 13.0, 8.09892 / 13.0],
     [9.47112 / 13.0, 4.84053 / 13.0],
     [11.2364 / 13.0, 10.0071 / 13.0]], dtype=np.float32)
_THRESH = 0.6
_PRIOR_ITER = 12800

_A = 5
_K = 50
_KP = 64
_HP = jax.lax.Precision.HIGHEST


def _per_truth(x1, y1, x2, y2, w_grid, h_grid):
    cw = x2 - x1
    ch = y2 - y1
    a1 = cw * ch
    best_iou = jnp.zeros_like(cw)
    best_ind = jnp.zeros_like(cw)
    for a in range(_A):
        aw = float(_ANCHORS[a, 0])
        ah = float(_ANCHORS[a, 1])
        a2 = float(np.float32(_ANCHORS[a, 0]) * np.float32(_ANCHORS[a, 1]))
        inter = jnp.minimum(cw, aw) * jnp.minimum(ch, ah)
        union = jnp.clip(a1 + a2 - inter, 1e-12, None)
        iou = inter / union
        upd = iou > best_iou
        best_ind = jnp.where(upd, float(a), best_ind)
        best_iou = jnp.where(upd, iou, best_iou)
    dx = (x1 + x2) / 2.0 * w_grid
    dy = (y1 + y2) / 2.0 * h_grid
    gxk = jnp.ceil(dx) - 1.0
    gyk = jnp.ceil(dy) - 1.0
    pos = gyk * w_grid + gxk
    return pos, best_ind, best_iou, cw, ch, dx - gxk, dy - gyk


def _prep_body(x_ref, t_ref, xt_ref, meta_ref, out_ref, *, grid_h, grid_w,
               t_ch, bpp):
    hw = grid_h * grid_w
    c_ch = t_ch // _A
    r = lax.broadcasted_iota(jnp.int32, (_A * 5, t_ch), 0)
    t = lax.broadcasted_iota(jnp.int32, (_A * 5, t_ch), 1)
    sel = (t == ((r // 5) * c_ch + r % 5)).astype(jnp.float32)
    eye5 = (lax.broadcasted_iota(jnp.int32, (5, 5), 0) ==
            lax.broadcasted_iota(jnp.int32, (5, 5), 1)).astype(jnp.float32)
    cell = lax.broadcasted_iota(jnp.int32, (1, hw), 1)
    gx = (cell % grid_w).astype(jnp.float32)
    gy = (cell // grid_w).astype(jnp.float32)
    cellf = cell.astype(jnp.float32)
    bf = jnp.bfloat16

    for bi in range(bpp):
        xraw = x_ref[bi]
        x = lax.dot_general(sel, xraw, (((1,), (1,)), ((), ())),
                            precision=_HP)
        xt_ref[bi] = x
        tb = t_ref[bi]
        tt = lax.dot_general(eye5, tb, (((1,), (1,)), ((), ())),
                             precision=_HP)
        pos, ind, miou, cw, ch, fx, fy = _per_truth(
            tt[0:1, :], tt[1:2, :], tt[2:3, :], tt[3:4, :],
            float(grid_w), float(grid_h))
        valid = miou != 0.0
        posj, indj, miouj = _per_truth(
            tb[:, 0:1], tb[:, 1:2], tb[:, 2:3], tb[:, 3:4],
            float(grid_w), float(grid_h))[:3]
        iota_j = lax.broadcasted_iota(jnp.int32, (_K, 1), 0)
        iota_m = lax.broadcasted_iota(jnp.int32, (1, _K), 1)
        beats = (posj == pos) & (indj == ind) & (
            (miouj > miou) | ((miouj == miou) & (iota_j < iota_m)))
        winner = valid & jnp.logical_not(jnp.any(beats, axis=0, keepdims=True))

        rdx = -jnp.log(1.0 / fx - 1.0)
        rdy = -jnp.log(1.0 / fy - 1.0)
        aw_sel = jnp.zeros_like(ind)
        ah_sel = jnp.zeros_like(ind)
        for a in range(_A):
            hit = ind == float(a)
            aw_sel = jnp.where(hit, float(_ANCHORS[a, 0]), aw_sel)
            ah_sel = jnp.where(hit, float(_ANCHORS[a, 1]), ah_sel)
        t2t = jnp.log(cw) / aw_sel
        t3t = jnp.log(ch) / ah_sel
        cf2 = 2.0 - cw * ch
        fb = pos + ind * float(5 * hw)

        zero = jnp.zeros_like(fb)
        rows = [jnp.where(winner, fb, zero), jnp.where(winner, cf2, zero),
                jnp.where(winner, rdx, zero), jnp.where(winner, rdy, zero),
                jnp.where(winner, t2t, zero), jnp.where(winner, t3t, zero),
                zero, zero]
        pad = jnp.zeros((8, _KP - _K), jnp.float32)
        meta_ref[bi] = jnp.concatenate(
            [jnp.concatenate(rows, axis=0), pad], axis=1)

        validj = miouj != 0.0
        tx1 = tb[:, 0:1].astype(bf)
        ty1 = tb[:, 1:2].astype(bf)
        tx2 = tb[:, 2:3].astype(bf)
        ty2 = tb[:, 3:4].astype(bf)
        a2t = (0.375 * ((tb[:, 2:3] - tb[:, 0:1]) *
                        (tb[:, 3:4] - tb[:, 1:2]))).astype(bf)
        acc_prior = jnp.float32(0.0)
        acc_noobj = jnp.float32(0.0)
        for a in range(_A):
            base = a * 5
            t0 = x[base + 0:base + 1, :]
            t1 = x[base + 1:base + 2, :]
            t2 = x[base + 2:base + 3, :]
            t3 = x[base + 3:base + 4, :]
            t4 = x[base + 4:base + 5, :]
            aw = float(_ANCHORS[a, 0])
            ah = float(_ANCHORS[a, 1])
            c0 = (1.0 / (1.0 + jnp.exp(-t0)) + gx) / float(grid_w)
            c1 = (1.0 / (1.0 + jnp.exp(-t1)) + gy) / float(grid_h)
            wa = jnp.exp(t2) * aw
            ha = jnp.exp(t3) * ah
            bx1 = c0 - wa / 2.0
            bx2 = c0 + wa / 2.0
            by1 = c1 - ha / 2.0
            by2 = c1 + ha / 2.0
            a1 = (bx2 - bx1) * (by2 - by1)
            acc_prior += jnp.sum((wa - aw) ** 2) + jnp.sum((ha - ah) ** 2)
            ixh = jnp.clip(jnp.minimum(bx2.astype(bf), tx2) -
                           jnp.maximum(bx1.astype(bf), tx1), 0.0, None)
            iyh = jnp.clip(jnp.minimum(by2.astype(bf), ty2) -
                           jnp.maximum(by1.astype(bf), ty1), 0.0, None)
            inter = ixh * iyh
            thr = (0.375 * a1).astype(bf) + a2t
            viol = jnp.max(inter - thr, axis=0, keepdims=True)
            noobj = viol.astype(jnp.float32) < 0.0
            match = (posj == cellf) & (indj == float(a)) & validj
            anymatch = jnp.any(match, axis=0, keepdims=True)
            acc_noobj += jnp.sum(
                jnp.where(noobj & jnp.logical_not(anymatch), t4, 0.0) ** 2)

        out_ref[bi] = jnp.concatenate(
            [acc_prior.reshape(1, 1), acc_noobj.reshape(1, 1)], axis=1)


def _make_sc_coord(b_total, elems_per_batch, ch_stride):
    info = plsc.get_sparse_core_info()
    nw = info.num_cores * info.num_subcores
    b_per_w = b_total // nw
    n_rows = 4 * b_per_w
    mesh = plsc.VectorSubcoreMesh(core_axis_name="c", subcore_axis_name="s")

    @functools.partial(
        pl.kernel, mesh=mesh,
        out_type=jax.ShapeDtypeStruct((b_total, 16), jnp.float32),
        scratch_types=[
            pltpu.VMEM((b_per_w, 8, _KP), jnp.float32),
            pltpu.VMEM((n_rows, _KP), jnp.int32),
            pltpu.VMEM((n_rows, _KP), jnp.float32),
            pltpu.VMEM((16,), jnp.float32),
            pltpu.SemaphoreType.DMA,
        ],
    )
    def sc_coord(xflat_hbm, meta_hbm, out_hbm, meta_v, idx_v, vals_v,
                 acc_v, sem):
        wid = lax.axis_index("s") * info.num_cores + lax.axis_index("c")
        base = wid * b_per_w
        for bi in range(b_per_w):
            pltpu.sync_copy(meta_hbm.at[base + bi], meta_v.at[bi])
        for bi in range(b_per_w):
            bb = (base + bi) * elems_per_batch
            for c in range(4):
                for chk in range(_KP // 16):
                    sl = pl.ds(chk * 16, 16)
                    fb = meta_v[bi, 0, sl]
                    idx_v[bi * 4 + c, sl] = (
                        fb.astype(jnp.int32) + (bb + c * ch_stride))
        cps = [pltpu.async_copy(xflat_hbm.at[idx_v.at[r]], vals_v.at[r], sem)
               for r in range(n_rows)]
        for cp in cps:
            cp.wait()
        for bi in range(b_per_w):
            acc = jnp.zeros((16,), jnp.float32)
            for chk in range(_KP // 16):
                sl = pl.ds(chk * 16, 16)
                wcf2 = meta_v[bi, 1, sl]
                d0 = vals_v[bi * 4 + 0, sl] - meta_v[bi, 2, sl]
                d1 = vals_v[bi * 4 + 1, sl] - meta_v[bi, 3, sl]
                d2 = vals_v[bi * 4 + 2, sl] - meta_v[bi, 4, sl]
                d3 = vals_v[bi * 4 + 3, sl] - meta_v[bi, 5, sl]
                acc = acc + wcf2 * (d0 * d0 + d1 * d1 + d2 * d2 + d3 * d3)
            acc_v[...] = acc
            pltpu.sync_copy(acc_v, out_hbm.at[base + bi])

    return sc_coord


def kernel(output, truths, iteration):
    b, grid_h, grid_w, t = output.shape
    hw = grid_h * grid_w
    k = truths.shape[1]
    ppp = 16

    x3 = output.reshape(b, hw, t)
    xt, meta, parts = pl.pallas_call(
        functools.partial(_prep_body, grid_h=grid_h, grid_w=grid_w,
                          t_ch=t, bpp=ppp),
        grid=(b // ppp,),
        in_specs=[
            pl.BlockSpec((ppp, hw, t), lambda i: (i, 0, 0)),
            pl.BlockSpec((ppp, k, 5), lambda i: (i, 0, 0)),
        ],
        out_specs=[
            pl.BlockSpec((ppp, _A * 5, hw), lambda i: (i, 0, 0)),
            pl.BlockSpec((ppp, 8, _KP), lambda i: (i, 0, 0)),
            pl.BlockSpec((ppp, 1, 2), lambda i: (i, 0, 0)),
        ],
        out_shape=[
            jax.ShapeDtypeStruct((b, _A * 5, hw), jnp.float32),
            jax.ShapeDtypeStruct((b, 8, _KP), jnp.float32),
            jax.ShapeDtypeStruct((b, 1, 2), jnp.float32),
        ],
    )(x3, truths)

    xtflat = xt.reshape(-1)
    coord_parts = _make_sc_coord(b, _A * 5 * hw, hw)(xtflat, meta)

    sums = jnp.sum(parts, axis=(0, 1))
    prior = jnp.where(iteration < _PRIOR_ITER, sums[0], jnp.float32(0.0))
    return prior + sums[1] + jnp.sum(coord_parts)


# speedup vs baseline: 1.9964x; 1.0641x over previous
import functools

import jax
import jax.numpy as jnp
import numpy as np
from jax import lax
from jax.experimental import pallas as pl
from jax.experimental.pallas import tpu as pltpu
from jax.experimental.pallas import tpu_sc as plsc

_ANCHORS = np.array(
    [[1.3221 / 13.0, 1.73145 / 13.0],
     [3.19275 / 13.0, 4.00944 / 13.0],
     [5.05587 / 13.0, 8.09892 / 13.0],
     [9.47112 / 13.0, 4.84053 / 13.0],
     [11.2364 / 13.0, 10.0071 / 13.0]], dtype=np.float32)
_THRESH = 0.6
_PRIOR_ITER = 12800

_A = 5
_K = 50
_KP = 64
_HP = jax.lax.Precision.HIGHEST


def _per_truth(x1, y1, x2, y2, w_grid, h_grid):
    cw = x2 - x1
    ch = y2 - y1
    a1 = cw * ch
    best_iou = jnp.zeros_like(cw)
    best_ind = jnp.zeros_like(cw)
    for a in range(_A):
        aw = float(_ANCHORS[a, 0])
        ah = float(_ANCHORS[a, 1])
        a2 = float(np.float32(_ANCHORS[a, 0]) * np.float32(_ANCHORS[a, 1]))
        inter = jnp.minimum(cw, aw) * jnp.minimum(ch, ah)
        union = jnp.clip(a1 + a2 - inter, 1e-12, None)
        iou = inter / union
        upd = iou > best_iou
        best_ind = jnp.where(upd, float(a), best_ind)
        best_iou = jnp.where(upd, iou, best_iou)
    dx = (x1 + x2) / 2.0 * w_grid
    dy = (y1 + y2) / 2.0 * h_grid
    gxk = jnp.ceil(dx) - 1.0
    gyk = jnp.ceil(dy) - 1.0
    pos = gyk * w_grid + gxk
    return pos, best_ind, best_iou, cw, ch, dx - gxk, dy - gyk


def _prep_body(x_ref, t_ref, xt_ref, meta_ref, out_ref, *, grid_h, grid_w,
               t_ch, bpp):
    hw = grid_h * grid_w
    c_ch = t_ch // _A
    r = lax.broadcasted_iota(jnp.int32, (_A * 5, t_ch), 0)
    t = lax.broadcasted_iota(jnp.int32, (_A * 5, t_ch), 1)
    sel = (t == ((r // 5) * c_ch + r % 5)).astype(jnp.float32)
    eye5 = (lax.broadcasted_iota(jnp.int32, (5, 5), 0) ==
            lax.broadcasted_iota(jnp.int32, (5, 5), 1)).astype(jnp.float32)
    cell = lax.broadcasted_iota(jnp.int32, (1, hw), 1)
    gx = (cell % grid_w).astype(jnp.float32)
    gy = (cell // grid_w).astype(jnp.float32)
    cellf = cell.astype(jnp.float32)
    bf = jnp.bfloat16

    for bi in range(bpp):
        xraw = x_ref[bi]
        x = lax.dot_general(sel, xraw, (((1,), (1,)), ((), ())),
                            precision=_HP)
        xt_ref[bi] = x
        tb = t_ref[bi]
        tt = lax.dot_general(eye5, tb, (((1,), (1,)), ((), ())),
                             precision=_HP)
        pos, ind, miou, cw, ch, fx, fy = _per_truth(
            tt[0:1, :], tt[1:2, :], tt[2:3, :], tt[3:4, :],
            float(grid_w), float(grid_h))
        valid = miou != 0.0
        posj, indj, miouj = _per_truth(
            tb[:, 0:1], tb[:, 1:2], tb[:, 2:3], tb[:, 3:4],
            float(grid_w), float(grid_h))[:3]
        iota_j = lax.broadcasted_iota(jnp.int32, (_K, 1), 0)
        iota_m = lax.broadcasted_iota(jnp.int32, (1, _K), 1)
        beats = (posj == pos) & (indj == ind) & (
            (miouj > miou) | ((miouj == miou) & (iota_j < iota_m)))
        winner = valid & jnp.logical_not(jnp.any(beats, axis=0, keepdims=True))

        rdx = -jnp.log(1.0 / fx - 1.0)
        rdy = -jnp.log(1.0 / fy - 1.0)
        aw_sel = jnp.zeros_like(ind)
        ah_sel = jnp.zeros_like(ind)
        for a in range(_A):
            hit = ind == float(a)
            aw_sel = jnp.where(hit, float(_ANCHORS[a, 0]), aw_sel)
            ah_sel = jnp.where(hit, float(_ANCHORS[a, 1]), ah_sel)
        t2t = jnp.log(cw) / aw_sel
        t3t = jnp.log(ch) / ah_sel
        cf2 = 2.0 - cw * ch
        fb = pos + ind * float(5 * hw)

        zero = jnp.zeros_like(fb)
        rows = [jnp.where(winner, fb, zero), jnp.where(winner, cf2, zero),
                jnp.where(winner, rdx, zero), jnp.where(winner, rdy, zero),
                jnp.where(winner, t2t, zero), jnp.where(winner, t3t, zero),
                zero, zero]
        pad = jnp.zeros((8, _KP - _K), jnp.float32)
        meta_ref[bi] = jnp.concatenate(
            [jnp.concatenate(rows, axis=0), pad], axis=1)

        validj = miouj != 0.0
        codej = jnp.where(validj, posj * 8.0 + indj, -1.0).astype(jnp.int32)
        cell8 = cell * 8
        tx1 = tb[:, 0:1].astype(bf)
        ty1 = tb[:, 1:2].astype(bf)
        tx2 = tb[:, 2:3].astype(bf)
        ty2 = tb[:, 3:4].astype(bf)
        a2t = (0.375 * ((tb[:, 2:3] - tb[:, 0:1]) *
                        (tb[:, 3:4] - tb[:, 1:2]))).astype(bf)
        acc_prior = jnp.float32(0.0)
        acc_noobj = jnp.float32(0.0)
        for a in range(_A):
            base = a * 5
            t0 = x[base + 0:base + 1, :]
            t1 = x[base + 1:base + 2, :]
            t2 = x[base + 2:base + 3, :]
            t3 = x[base + 3:base + 4, :]
            t4 = x[base + 4:base + 5, :]
            aw = float(_ANCHORS[a, 0])
            ah = float(_ANCHORS[a, 1])
            c0 = (1.0 / (1.0 + jnp.exp(-t0)) + gx) / float(grid_w)
            c1 = (1.0 / (1.0 + jnp.exp(-t1)) + gy) / float(grid_h)
            wa = jnp.exp(t2) * aw
            ha = jnp.exp(t3) * ah
            bx1 = c0 - wa / 2.0
            bx2 = c0 + wa / 2.0
            by1 = c1 - ha / 2.0
            by2 = c1 + ha / 2.0
            a1 = (bx2 - bx1) * (by2 - by1)
            acc_prior += jnp.sum((wa - aw) ** 2) + jnp.sum((ha - ah) ** 2)
            ixh = jnp.clip(jnp.minimum(bx2.astype(bf), tx2) -
                           jnp.maximum(bx1.astype(bf), tx1), 0.0, None)
            iyh = jnp.clip(jnp.minimum(by2.astype(bf), ty2) -
                           jnp.maximum(by1.astype(bf), ty1), 0.0, None)
            inter = ixh * iyh
            thr = (0.375 * a1).astype(bf) + a2t
            viol = jnp.max(inter - thr, axis=0, keepdims=True)
            noobj = viol.astype(jnp.float32) < 0.0
            eqv = jnp.where(codej == cell8 + a, 1, 0)
            anymatch = jnp.max(eqv, axis=0, keepdims=True) > 0
            acc_noobj += jnp.sum(
                jnp.where(noobj & jnp.logical_not(anymatch), t4, 0.0) ** 2)

        out_ref[bi] = jnp.concatenate(
            [acc_prior.reshape(1, 1), acc_noobj.reshape(1, 1)], axis=1)


def _make_sc_coord(b_total, elems_per_batch, ch_stride):
    info = plsc.get_sparse_core_info()
    nw = info.num_cores * info.num_subcores
    b_per_w = b_total // nw
    n_rows = 4 * b_per_w
    mesh = plsc.VectorSubcoreMesh(core_axis_name="c", subcore_axis_name="s")

    @functools.partial(
        pl.kernel, mesh=mesh,
        out_type=jax.ShapeDtypeStruct((b_total, 16), jnp.float32),
        scratch_types=[
            pltpu.VMEM((b_per_w, 8, _KP), jnp.float32),
            pltpu.VMEM((n_rows, _KP), jnp.int32),
            pltpu.VMEM((n_rows, _KP), jnp.float32),
            pltpu.VMEM((16,), jnp.float32),
            pltpu.SemaphoreType.DMA,
        ],
    )
    def sc_coord(xflat_hbm, meta_hbm, out_hbm, meta_v, idx_v, vals_v,
                 acc_v, sem):
        wid = lax.axis_index("s") * info.num_cores + lax.axis_index("c")
        base = wid * b_per_w
        for bi in range(b_per_w):
            pltpu.sync_copy(meta_hbm.at[base + bi], meta_v.at[bi])
        for bi in range(b_per_w):
            bb = (base + bi) * elems_per_batch
            for c in range(4):
                for chk in range(_KP // 16):
                    sl = pl.ds(chk * 16, 16)
                    fb = meta_v[bi, 0, sl]
                    idx_v[bi * 4 + c, sl] = (
                        fb.astype(jnp.int32) + (bb + c * ch_stride))
        cps = [pltpu.async_copy(xflat_hbm.at[idx_v.at[r]], vals_v.at[r], sem)
               for r in range(n_rows)]
        for cp in cps:
            cp.wait()
        for bi in range(b_per_w):
            acc = jnp.zeros((16,), jnp.float32)
            for chk in range(_KP // 16):
                sl = pl.ds(chk * 16, 16)
                wcf2 = meta_v[bi, 1, sl]
                d0 = vals_v[bi * 4 + 0, sl] - meta_v[bi, 2, sl]
                d1 = vals_v[bi * 4 + 1, sl] - meta_v[bi, 3, sl]
                d2 = vals_v[bi * 4 + 2, sl] - meta_v[bi, 4, sl]
                d3 = vals_v[bi * 4 + 3, sl] - meta_v[bi, 5, sl]
                acc = acc + wcf2 * (d0 * d0 + d1 * d1 + d2 * d2 + d3 * d3)
            acc_v[...] = acc
            pltpu.sync_copy(acc_v, out_hbm.at[base + bi])

    return sc_coord


def kernel(output, truths, iteration):
    b, grid_h, grid_w, t = output.shape
    hw = grid_h * grid_w
    k = truths.shape[1]
    ppp = 8

    x3 = output.reshape(b, hw, t)
    xt, meta, parts = pl.pallas_call(
        functools.partial(_prep_body, grid_h=grid_h, grid_w=grid_w,
                          t_ch=t, bpp=ppp),
        grid=(b // ppp,),
        in_specs=[
            pl.BlockSpec((ppp, hw, t), lambda i: (i, 0, 0)),
            pl.BlockSpec((ppp, k, 5), lambda i: (i, 0, 0)),
        ],
        out_specs=[
            pl.BlockSpec((ppp, _A * 5, hw), lambda i: (i, 0, 0)),
            pl.BlockSpec((ppp, 8, _KP), lambda i: (i, 0, 0)),
            pl.BlockSpec((ppp, 1, 2), lambda i: (i, 0, 0)),
        ],
        out_shape=[
            jax.ShapeDtypeStruct((b, _A * 5, hw), jnp.float32),
            jax.ShapeDtypeStruct((b, 8, _KP), jnp.float32),
            jax.ShapeDtypeStruct((b, 1, 2), jnp.float32),
        ],
    )(x3, truths)

    xtflat = xt.reshape(-1)
    coord_parts = _make_sc_coord(b, _A * 5 * hw, hw)(xtflat, meta)

    sums = jnp.sum(parts, axis=(0, 1))
    prior = jnp.where(iteration < _PRIOR_ITER, sums[0], jnp.float32(0.0))
    return prior + sums[1] + jnp.sum(coord_parts)
